# baseline (speedup 1.0000x reference)
import jax
import jax.numpy as jnp
from jax import lax
from jax.experimental import pallas as pl
from jax.experimental.pallas import tpu as pltpu
from jax.experimental.pallas import tpu_sc as plsc

N = 10000
E = 320000
D = 128

NC = 2
NS = 16
NW = NC * NS
K = 56
CH = 184
EPT = CH * K
EPAD = NW * EPT
NPAD = 10112
RS = NPAD // NS
DW = 16

_mesh = plsc.VectorSubcoreMesh(core_axis_name="c", subcore_axis_name="s")


def _agg_body(g_hbm, srcs_hbm, dsts_hbm, zrows_hbm, acc_out,
              src_v, dst_v, rows_0, rows_1, rows_2, rows_3, acc,
              gsem_0, gsem_1, gsem_2, gsem_3,
              ssem_0, ssem_1, ssem_2, ssem_3):
    c = lax.axis_index("c")
    s = lax.axis_index("s")
    wid = c * NS + s
    bufs = (rows_0, rows_1, rows_2, rows_3)
    gsems = (gsem_0, gsem_1, gsem_2, gsem_3)
    ssems = (ssem_0, ssem_1, ssem_2, ssem_3)

    pltpu.sync_copy(zrows_hbm.at[pl.ds(s * RS, RS)], acc.at[pl.ds(s * RS, RS)])
    pltpu.sync_copy(srcs_hbm.at[wid], src_v)
    pltpu.sync_copy(dsts_hbm.at[wid], dst_v)
    plsc.subcore_barrier()

    def gather(t, b):
        return pltpu.make_async_copy(
            g_hbm.at[src_v.at[pl.ds(t * K, K)]], bufs[b], gsems[b])

    def scatter_wait(b):
        pltpu.make_async_copy(
            bufs[b], acc.at[dst_v.at[pl.ds(0, K)]], ssems[b]).wait()

    gather(0, 0).start()
    gather(1, 1).start()

    def step(p, carry):
        t0 = 4 * p
        for i in range(4):
            t = t0 + i
            b = i
            b2 = (i + 2) % 4
            gather(t, b).wait()
            pltpu.async_copy(bufs[b], acc.at[dst_v.at[pl.ds(t * K, K)]],
                             ssems[b], add=True)

            @pl.when(t + 2 < CH)
            def _():
                @pl.when(t >= 2)
                def _():
                    scatter_wait(b2)

                gather(t + 2, b2).start()

        return carry

    lax.fori_loop(0, CH // 4, step, 0)
    scatter_wait(0)
    scatter_wait(1)
    scatter_wait(2)
    scatter_wait(3)
    plsc.subcore_barrier()

    pltpu.sync_copy(acc.at[pl.ds(s * RS, RS)],
                    acc_out.at[c].at[pl.ds(s * RS, RS)])


_sc_agg = pl.kernel(
    _agg_body,
    mesh=_mesh,
    out_type=[jax.ShapeDtypeStruct((NC, NPAD, D), jnp.float32)],
    scratch_types=[
        pltpu.VMEM((EPT,), jnp.int32),
        pltpu.VMEM((EPT,), jnp.int32),
        pltpu.VMEM((K, D), jnp.float32),
        pltpu.VMEM((K, D), jnp.float32),
        pltpu.VMEM((K, D), jnp.float32),
        pltpu.VMEM((K, D), jnp.float32),
        pltpu.VMEM_SHARED((NPAD, D), jnp.float32),
        pltpu.SemaphoreType.DMA,
        pltpu.SemaphoreType.DMA,
        pltpu.SemaphoreType.DMA,
        pltpu.SemaphoreType.DMA,
        pltpu.SemaphoreType.DMA,
        pltpu.SemaphoreType.DMA,
        pltpu.SemaphoreType.DMA,
        pltpu.SemaphoreType.DMA,
    ],
)


NDEG = 10240
ZR = NDEG // NS


def _deg_body(dsts_hbm, ones_hbm, deg_out, dst_v, ones_v, tmp_v, big_v, dacc):
    c = lax.axis_index("c")
    s = lax.axis_index("s")
    wid = c * NS + s

    def zstep(i, carry):
        tmp_v[pl.ds(i * 16, 16)] = jnp.zeros((16,), jnp.float32)
        return carry

    lax.fori_loop(0, ZR // 16, zstep, 0)
    pltpu.sync_copy(tmp_v, dacc.at[pl.ds(s * ZR, ZR)])
    pltpu.sync_copy(ones_hbm, ones_v)
    pltpu.sync_copy(dsts_hbm.at[wid], dst_v)
    plsc.subcore_barrier()

    def step(j, carry):
        pltpu.sync_copy(ones_v, dacc.at[dst_v.at[j]], add=True)
        return carry

    lax.fori_loop(0, CH, step, 0)
    plsc.subcore_barrier()

    @pl.when(s == 0)
    def _():
        pltpu.sync_copy(dacc, big_v)
        pltpu.sync_copy(big_v, deg_out.at[c * 8])


_sc_deg = pl.kernel(
    _deg_body,
    mesh=_mesh,
    out_type=[jax.ShapeDtypeStruct((16, NDEG), jnp.float32)],
    scratch_types=[
        pltpu.VMEM((CH, K), jnp.int32),
        pltpu.VMEM((K,), jnp.float32),
        pltpu.VMEM((ZR,), jnp.float32),
        pltpu.VMEM((NDEG,), jnp.float32),
        pltpu.VMEM_SHARED((NDEG,), jnp.float32),
    ],
)

_ROWS_BLK = 2000
_GRID = N // _ROWS_BLK


def _tc1_body(x_ref, wn_ref, ws_ref, b_ref, g_ref, s_ref):
    xb = x_ref[...]
    g_ref[...] = jnp.dot(xb, wn_ref[...], preferred_element_type=jnp.float32)
    s_ref[...] = jnp.dot(xb, ws_ref[...],
                         preferred_element_type=jnp.float32) + b_ref[...]


def _tc2_body(s1_ref, a0_ref, a1_ref, d0_ref, d1_ref, wn_ref, ws_ref, b_ref,
              g_ref, s_ref):
    deg = d0_ref[...] + d1_ref[...]
    rdeg = 1.0 / jnp.maximum(deg, 1.0)
    h = jnp.maximum(s1_ref[...] + (a0_ref[...] + a1_ref[...]) * rdeg, 0.0)
    g_ref[...] = jnp.dot(h, wn_ref[...], preferred_element_type=jnp.float32)
    s_ref[...] = jnp.dot(h, ws_ref[...],
                         preferred_element_type=jnp.float32) + b_ref[...]


def _tc3_body(s2_ref, a0_ref, a1_ref, d0_ref, d1_ref, o_ref):
    deg = d0_ref[...] + d1_ref[...]
    rdeg = 1.0 / jnp.maximum(deg, 1.0)
    o_ref[...] = s2_ref[...] + (a0_ref[...] + a1_ref[...]) * rdeg


def _row_spec(w):
    return pl.BlockSpec((_ROWS_BLK, w), lambda i: (i, 0))


def _full_spec(h, w):
    return pl.BlockSpec((h, w), lambda i: (0, 0))


_tc1 = pl.pallas_call(
    _tc1_body,
    grid=(_GRID,),
    in_specs=[_row_spec(D), _full_spec(D, D), _full_spec(D, D),
              _full_spec(1, D)],
    out_specs=[_row_spec(D), _row_spec(D)],
    out_shape=[jax.ShapeDtypeStruct((N, D), jnp.float32)] * 2,
)

_tc2 = pl.pallas_call(
    _tc2_body,
    grid=(_GRID,),
    in_specs=[_row_spec(D), _row_spec(D), _row_spec(D), _row_spec(1),
              _row_spec(1), _full_spec(D, D), _full_spec(D, D),
              _full_spec(1, D)],
    out_specs=[_row_spec(D), _row_spec(D)],
    out_shape=[jax.ShapeDtypeStruct((N, D), jnp.float32)] * 2,
)

_tc3 = pl.pallas_call(
    _tc3_body,
    grid=(_GRID,),
    in_specs=[_row_spec(D), _row_spec(D), _row_spec(D), _row_spec(1),
              _row_spec(1)],
    out_specs=_row_spec(D),
    out_shape=jax.ShapeDtypeStruct((N, D), jnp.float32),
)


def kernel(x, edge_index, W_neigh1, W_self1, b1, W_neigh2, W_self2, b2):
    ei = edge_index.astype(jnp.int32)
    ppw = EPT - E // NW
    src_p = jnp.concatenate(
        [ei[0].reshape(NW, E // NW),
         jnp.zeros((NW, ppw), jnp.int32)], axis=1).reshape(NW, EPT)
    dst_p = jnp.concatenate(
        [ei[1].reshape(NW, E // NW),
         jnp.full((NW, ppw), N, jnp.int32)], axis=1).reshape(NW, EPT)
    dst_p3 = dst_p.reshape(NW, CH, K)
    zrows = jnp.zeros((NPAD, D), jnp.float32)
    ones = jnp.ones((K,), jnp.float32)

    (degv,) = _sc_deg(dst_p3, ones)
    d0 = degv[0, :N].reshape(N, 1)
    d1 = degv[8, :N].reshape(N, 1)
    g1, s1 = _tc1(x, W_neigh1, W_self1, b1[None, :])
    (acc1,) = _sc_agg(g1, src_p, dst_p, zrows)
    g2, s2 = _tc2(s1, acc1[0, :N], acc1[1, :N], d0, d1,
                  W_neigh2, W_self2, b2[None, :])
    (acc2,) = _sc_agg(g2, src_p, dst_p, zrows)
    return _tc3(s2, acc2[0, :N], acc2[1, :N], d0, d1)


# speedup vs baseline: 1.3974x; 1.3974x over previous
import jax
import jax.numpy as jnp
from jax import lax
from jax.experimental import pallas as pl
from jax.experimental.pallas import tpu as pltpu
from jax.experimental.pallas import tpu_sc as plsc

N = 10000
E = 320000
D = 128

NC = 2
NS = 16
NW = NC * NS
K = 72
CH = 141
EPT = CH * K
EPAD = NW * EPT
NPAD = 10112
RS = NPAD // NS
DW = 16

_mesh = plsc.VectorSubcoreMesh(core_axis_name="c", subcore_axis_name="s")


def _agg_body(g_hbm, srcs_hbm, dsts_hbm, zrows_hbm, acc_out,
              src_v, dst_v, rows_0, rows_1, rows_2, acc,
              gsem_0, gsem_1, gsem_2, ssem_0, ssem_1, ssem_2):
    c = lax.axis_index("c")
    s = lax.axis_index("s")
    wid = c * NS + s
    bufs = (rows_0, rows_1, rows_2)
    gsems = (gsem_0, gsem_1, gsem_2)
    ssems = (ssem_0, ssem_1, ssem_2)

    pltpu.sync_copy(zrows_hbm.at[pl.ds(s * RS, RS)], acc.at[pl.ds(s * RS, RS)])
    pltpu.sync_copy(srcs_hbm.at[wid], src_v)
    pltpu.sync_copy(dsts_hbm.at[wid], dst_v)
    plsc.subcore_barrier()

    def gather(t, b):
        return pltpu.make_async_copy(
            g_hbm.at[src_v.at[pl.ds(t * K, K)]], bufs[b], gsems[b])

    def scatter_wait(b):
        pltpu.make_async_copy(
            bufs[b], acc.at[dst_v.at[pl.ds(0, K)]], ssems[b]).wait()

    gather(0, 0).start()
    gather(1, 1).start()
    gather(2, 2).start()

    def step(p, carry):
        t0 = 3 * p
        for i in range(3):
            t = t0 + i
            b = i
            gather(t, b).wait()
            pltpu.async_copy(bufs[b], acc.at[dst_v.at[pl.ds(t * K, K)]],
                             ssems[b], add=True)
            nxt = t + 1
            b1 = (i + 1) % 3

            @pl.when((nxt >= 3) & (nxt < CH))
            def _():
                scatter_wait(b1)
                gather(nxt, b1).start()

        return carry

    lax.fori_loop(0, CH // 3, step, 0)
    scatter_wait(0)
    scatter_wait(1)
    scatter_wait(2)
    plsc.subcore_barrier()

    pltpu.sync_copy(acc.at[pl.ds(s * RS, RS)],
                    acc_out.at[c].at[pl.ds(s * RS, RS)])


_sc_agg = pl.kernel(
    _agg_body,
    mesh=_mesh,
    out_type=[jax.ShapeDtypeStruct((NC, NPAD, D), jnp.float32)],
    scratch_types=[
        pltpu.VMEM((EPT,), jnp.int32),
        pltpu.VMEM((EPT,), jnp.int32),
        pltpu.VMEM((K, D), jnp.float32),
        pltpu.VMEM((K, D), jnp.float32),
        pltpu.VMEM((K, D), jnp.float32),
        pltpu.VMEM_SHARED((NPAD, D), jnp.float32),
        pltpu.SemaphoreType.DMA,
        pltpu.SemaphoreType.DMA,
        pltpu.SemaphoreType.DMA,
        pltpu.SemaphoreType.DMA,
        pltpu.SemaphoreType.DMA,
        pltpu.SemaphoreType.DMA,
    ],
)


NDEG = 10240
ZR = NDEG // NS


def _deg_body(dsts_hbm, ones_hbm, deg_out, dst_v, ones_v, tmp_v, big_v, dacc):
    c = lax.axis_index("c")
    s = lax.axis_index("s")
    wid = c * NS + s

    def zstep(i, carry):
        tmp_v[pl.ds(i * 16, 16)] = jnp.zeros((16,), jnp.float32)
        return carry

    lax.fori_loop(0, ZR // 16, zstep, 0)
    pltpu.sync_copy(tmp_v, dacc.at[pl.ds(s * ZR, ZR)])
    pltpu.sync_copy(ones_hbm, ones_v)
    pltpu.sync_copy(dsts_hbm.at[wid], dst_v)
    plsc.subcore_barrier()

    def step(j, carry):
        pltpu.sync_copy(ones_v, dacc.at[dst_v.at[j]], add=True)
        return carry

    lax.fori_loop(0, CH, step, 0)
    plsc.subcore_barrier()

    @pl.when(s == 0)
    def _():
        pltpu.sync_copy(dacc, big_v)
        pltpu.sync_copy(big_v, deg_out.at[c * 8])


_sc_deg = pl.kernel(
    _deg_body,
    mesh=_mesh,
    out_type=[jax.ShapeDtypeStruct((16, NDEG), jnp.float32)],
    scratch_types=[
        pltpu.VMEM((CH, K), jnp.int32),
        pltpu.VMEM((K,), jnp.float32),
        pltpu.VMEM((ZR,), jnp.float32),
        pltpu.VMEM((NDEG,), jnp.float32),
        pltpu.VMEM_SHARED((NDEG,), jnp.float32),
    ],
)

_ROWS_BLK = 2000
_GRID = N // _ROWS_BLK


def _tc1_body(x_ref, wn_ref, ws_ref, b_ref, g_ref, s_ref):
    xb = x_ref[...]
    g_ref[...] = jnp.dot(xb, wn_ref[...], preferred_element_type=jnp.float32)
    s_ref[...] = jnp.dot(xb, ws_ref[...],
                         preferred_element_type=jnp.float32) + b_ref[...]


def _tc2_body(s1_ref, a0_ref, a1_ref, d0_ref, d1_ref, wn_ref, ws_ref, b_ref,
              g_ref, s_ref):
    deg = d0_ref[...] + d1_ref[...]
    rdeg = 1.0 / jnp.maximum(deg, 1.0)
    h = jnp.maximum(s1_ref[...] + (a0_ref[...] + a1_ref[...]) * rdeg, 0.0)
    g_ref[...] = jnp.dot(h, wn_ref[...], preferred_element_type=jnp.float32)
    s_ref[...] = jnp.dot(h, ws_ref[...],
                         preferred_element_type=jnp.float32) + b_ref[...]


def _tc3_body(s2_ref, a0_ref, a1_ref, d0_ref, d1_ref, o_ref):
    deg = d0_ref[...] + d1_ref[...]
    rdeg = 1.0 / jnp.maximum(deg, 1.0)
    o_ref[...] = s2_ref[...] + (a0_ref[...] + a1_ref[...]) * rdeg


def _row_spec(w):
    return pl.BlockSpec((_ROWS_BLK, w), lambda i: (i, 0))


def _full_spec(h, w):
    return pl.BlockSpec((h, w), lambda i: (0, 0))


_tc1 = pl.pallas_call(
    _tc1_body,
    grid=(_GRID,),
    in_specs=[_row_spec(D), _full_spec(D, D), _full_spec(D, D),
              _full_spec(1, D)],
    out_specs=[_row_spec(D), _row_spec(D)],
    out_shape=[jax.ShapeDtypeStruct((N, D), jnp.float32)] * 2,
)

_tc2 = pl.pallas_call(
    _tc2_body,
    grid=(_GRID,),
    in_specs=[_row_spec(D), _row_spec(D), _row_spec(D), _row_spec(1),
              _row_spec(1), _full_spec(D, D), _full_spec(D, D),
              _full_spec(1, D)],
    out_specs=[_row_spec(D), _row_spec(D)],
    out_shape=[jax.ShapeDtypeStruct((N, D), jnp.float32)] * 2,
)

_tc3 = pl.pallas_call(
    _tc3_body,
    grid=(_GRID,),
    in_specs=[_row_spec(D), _row_spec(D), _row_spec(D), _row_spec(1),
              _row_spec(1)],
    out_specs=_row_spec(D),
    out_shape=jax.ShapeDtypeStruct((N, D), jnp.float32),
)


def kernel(x, edge_index, W_neigh1, W_self1, b1, W_neigh2, W_self2, b2):
    ei = edge_index.astype(jnp.int32)
    ppw = EPT - E // NW
    src_p = jnp.concatenate(
        [ei[0].reshape(NW, E // NW),
         jnp.zeros((NW, ppw), jnp.int32)], axis=1).reshape(NW, EPT)
    dst_p = jnp.concatenate(
        [ei[1].reshape(NW, E // NW),
         jnp.full((NW, ppw), N, jnp.int32)], axis=1).reshape(NW, EPT)
    dst_p3 = dst_p.reshape(NW, CH, K)
    zrows = jnp.zeros((NPAD, D), jnp.float32)
    ones = jnp.ones((K,), jnp.float32)

    (degv,) = _sc_deg(dst_p3, ones)
    d0 = degv[0, :N].reshape(N, 1)
    d1 = degv[8, :N].reshape(N, 1)
    g1, s1 = _tc1(x, W_neigh1, W_self1, b1[None, :])
    (acc1,) = _sc_agg(g1, src_p, dst_p, zrows)
    g2, s2 = _tc2(s1, acc1[0, :N], acc1[1, :N], d0, d1,
                  W_neigh2, W_self2, b2[None, :])
    (acc2,) = _sc_agg(g2, src_p, dst_p, zrows)
    return _tc3(s2, acc2[0, :N], acc2[1, :N], d0, d1)


# speedup vs baseline: 1.9855x; 1.4209x over previous
import jax
import jax.numpy as jnp
from jax import lax
from jax.experimental import pallas as pl
from jax.experimental.pallas import tpu as pltpu
from jax.experimental.pallas import tpu_sc as plsc

N = 10000
E = 320000
D = 128

NC = 2
NS = 16
NW = NC * NS
K = 112
CH = 90
EPT = CH * K
EPAD = NW * EPT
NPAD = 10112
RS = NPAD // NS
DW = 16

_mesh = plsc.VectorSubcoreMesh(core_axis_name="c", subcore_axis_name="s")


def _agg_body(g_hbm, srcs_hbm, dsts_hbm, zrows_hbm, acc_out,
              src_v, dst_v, rows_0, rows_1, acc,
              gsem_0, gsem_1, ssem_0, ssem_1):
    c = lax.axis_index("c")
    s = lax.axis_index("s")
    wid = c * NS + s
    bufs = (rows_0, rows_1)
    gsems = (gsem_0, gsem_1)
    ssems = (ssem_0, ssem_1)

    pltpu.sync_copy(zrows_hbm.at[pl.ds(s * RS, RS)], acc.at[pl.ds(s * RS, RS)])
    pltpu.sync_copy(srcs_hbm.at[wid], src_v)
    pltpu.sync_copy(dsts_hbm.at[wid], dst_v)
    plsc.subcore_barrier()

    def gather(t, b):
        return pltpu.make_async_copy(
            g_hbm.at[src_v.at[pl.ds(t * K, K)]], bufs[b], gsems[b])

    def scatter_wait(b):
        pltpu.make_async_copy(
            bufs[b], acc.at[dst_v.at[pl.ds(0, K)]], ssems[b]).wait()

    gather(0, 0).start()
    gather(1, 1).start()

    def step(p, carry):
        t0 = 2 * p
        for i in range(2):
            t = t0 + i
            b = i
            b1 = (i + 1) % 2
            gather(t, b).wait()
            pltpu.async_copy(bufs[b], acc.at[dst_v.at[pl.ds(t * K, K)]],
                             ssems[b], add=True)
            nxt = t + 1

            @pl.when((nxt >= 2) & (nxt < CH))
            def _():
                scatter_wait(b1)
                gather(nxt, b1).start()

        return carry

    lax.fori_loop(0, CH // 2, step, 0)
    scatter_wait(0)
    scatter_wait(1)
    plsc.subcore_barrier()

    pltpu.sync_copy(acc.at[pl.ds(s * RS, RS)],
                    acc_out.at[c].at[pl.ds(s * RS, RS)])


_sc_agg = pl.kernel(
    _agg_body,
    mesh=_mesh,
    out_type=[jax.ShapeDtypeStruct((NC, NPAD, D), jnp.float32)],
    scratch_types=[
        pltpu.VMEM((EPT,), jnp.int32),
        pltpu.VMEM((EPT,), jnp.int32),
        pltpu.VMEM((K, D), jnp.float32),
        pltpu.VMEM((K, D), jnp.float32),
        pltpu.VMEM_SHARED((NPAD, D), jnp.float32),
        pltpu.SemaphoreType.DMA,
        pltpu.SemaphoreType.DMA,
        pltpu.SemaphoreType.DMA,
        pltpu.SemaphoreType.DMA,
    ],
)


NDEG = 10240
ZR = NDEG // NS


def _deg_body(dsts_hbm, ones_hbm, deg_out, dst_v, ones_v, tmp_v, big_v, dacc):
    c = lax.axis_index("c")
    s = lax.axis_index("s")
    wid = c * NS + s

    def zstep(i, carry):
        tmp_v[pl.ds(i * 16, 16)] = jnp.zeros((16,), jnp.float32)
        return carry

    lax.fori_loop(0, ZR // 16, zstep, 0)
    pltpu.sync_copy(tmp_v, dacc.at[pl.ds(s * ZR, ZR)])
    pltpu.sync_copy(ones_hbm, ones_v)
    pltpu.sync_copy(dsts_hbm.at[wid], dst_v)
    plsc.subcore_barrier()

    def step(j, carry):
        pltpu.sync_copy(ones_v, dacc.at[dst_v.at[j]], add=True)
        return carry

    lax.fori_loop(0, CH, step, 0)
    plsc.subcore_barrier()

    @pl.when(s == 0)
    def _():
        pltpu.sync_copy(dacc, big_v)
        pltpu.sync_copy(big_v, deg_out.at[c * 8])


_sc_deg = pl.kernel(
    _deg_body,
    mesh=_mesh,
    out_type=[jax.ShapeDtypeStruct((16, NDEG), jnp.float32)],
    scratch_types=[
        pltpu.VMEM((CH, K), jnp.int32),
        pltpu.VMEM((K,), jnp.float32),
        pltpu.VMEM((ZR,), jnp.float32),
        pltpu.VMEM((NDEG,), jnp.float32),
        pltpu.VMEM_SHARED((NDEG,), jnp.float32),
    ],
)

_ROWS_BLK = 2000
_GRID = N // _ROWS_BLK


def _tc1_body(x_ref, wn_ref, ws_ref, b_ref, g_ref, s_ref):
    xb = x_ref[...]
    g_ref[...] = jnp.dot(xb, wn_ref[...], preferred_element_type=jnp.float32)
    s_ref[...] = jnp.dot(xb, ws_ref[...],
                         preferred_element_type=jnp.float32) + b_ref[...]


def _tc2_body(s1_ref, a0_ref, a1_ref, d0_ref, d1_ref, wn_ref, ws_ref, b_ref,
              g_ref, s_ref):
    deg = d0_ref[...] + d1_ref[...]
    rdeg = 1.0 / jnp.maximum(deg, 1.0)
    h = jnp.maximum(s1_ref[...] + (a0_ref[...] + a1_ref[...]) * rdeg, 0.0)
    g_ref[...] = jnp.dot(h, wn_ref[...], preferred_element_type=jnp.float32)
    s_ref[...] = jnp.dot(h, ws_ref[...],
                         preferred_element_type=jnp.float32) + b_ref[...]


def _tc3_body(s2_ref, a0_ref, a1_ref, d0_ref, d1_ref, o_ref):
    deg = d0_ref[...] + d1_ref[...]
    rdeg = 1.0 / jnp.maximum(deg, 1.0)
    o_ref[...] = s2_ref[...] + (a0_ref[...] + a1_ref[...]) * rdeg


def _row_spec(w):
    return pl.BlockSpec((_ROWS_BLK, w), lambda i: (i, 0))


def _full_spec(h, w):
    return pl.BlockSpec((h, w), lambda i: (0, 0))


_tc1 = pl.pallas_call(
    _tc1_body,
    grid=(_GRID,),
    in_specs=[_row_spec(D), _full_spec(D, D), _full_spec(D, D),
              _full_spec(1, D)],
    out_specs=[_row_spec(D), _row_spec(D)],
    out_shape=[jax.ShapeDtypeStruct((N, D), jnp.float32)] * 2,
)

_tc2 = pl.pallas_call(
    _tc2_body,
    grid=(_GRID,),
    in_specs=[_row_spec(D), _row_spec(D), _row_spec(D), _row_spec(1),
              _row_spec(1), _full_spec(D, D), _full_spec(D, D),
              _full_spec(1, D)],
    out_specs=[_row_spec(D), _row_spec(D)],
    out_shape=[jax.ShapeDtypeStruct((N, D), jnp.float32)] * 2,
)

_tc3 = pl.pallas_call(
    _tc3_body,
    grid=(_GRID,),
    in_specs=[_row_spec(D), _row_spec(D), _row_spec(D), _row_spec(1),
              _row_spec(1)],
    out_specs=_row_spec(D),
    out_shape=jax.ShapeDtypeStruct((N, D), jnp.float32),
)


def kernel(x, edge_index, W_neigh1, W_self1, b1, W_neigh2, W_self2, b2):
    ei = edge_index.astype(jnp.int32)
    ppw = EPT - E // NW
    src_p = jnp.concatenate(
        [ei[0].reshape(NW, E // NW),
         jnp.zeros((NW, ppw), jnp.int32)], axis=1).reshape(NW, EPT)
    dst_p = jnp.concatenate(
        [ei[1].reshape(NW, E // NW),
         jnp.full((NW, ppw), N, jnp.int32)], axis=1).reshape(NW, EPT)
    dst_p3 = dst_p.reshape(NW, CH, K)
    zrows = jnp.zeros((NPAD, D), jnp.float32)
    ones = jnp.ones((K,), jnp.float32)

    (degv,) = _sc_deg(dst_p3, ones)
    d0 = degv[0, :N].reshape(N, 1)
    d1 = degv[8, :N].reshape(N, 1)
    g1, s1 = _tc1(x, W_neigh1, W_self1, b1[None, :])
    (acc1,) = _sc_agg(g1, src_p, dst_p, zrows)
    g2, s2 = _tc2(s1, acc1[0, :N], acc1[1, :N], d0, d1,
                  W_neigh2, W_self2, b2[None, :])
    (acc2,) = _sc_agg(g2, src_p, dst_p, zrows)
    return _tc3(s2, acc2[0, :N], acc2[1, :N], d0, d1)
